# Initial kernel scaffold; baseline (speedup 1.0000x reference)
import jax
import jax.numpy as jnp
from jax.experimental import pallas as pl


def kernel(x, edge_index, edge_attr, u, batch, W1a, b1a, W1b, b1b, W2a, b2a, W2b, b2b):
    raise NotImplementedError("write your pallas kernel here")


# speedup vs baseline: 1.9466x; 1.9466x over previous
import functools

import jax
import jax.numpy as jnp
from jax import lax
from jax.experimental import pallas as pl
from jax.experimental.pallas import tpu as pltpu
from jax.experimental.pallas import tpu_sc as plsc

_N = 10000
_E = 320000
_D = 128
_DE = 16
_DU = 64
_H = 64
_DOUT = 128
_B = 16

_NP = 10240
_EP = 327680

_NC = 2
_NS = 16
_NW = _NC * _NS
_CHUNK = 128
_EPT = _EP // _NW
_STEPS = _EPT // _CHUNK
_WIDE = _H + 16
_RPT = _NP // _NS


def _xa_body(x_ref, w_ref, o_ref):
    o_ref[...] = jnp.dot(x_ref[...], w_ref[...],
                         preferred_element_type=jnp.float32)


def _prep_xa(x_pad, w1a_x):
    return pl.pallas_call(
        _xa_body,
        out_shape=jax.ShapeDtypeStruct((_NP, _H), jnp.float32),
    )(x_pad, w1a_x)


_BE = 8192


def _ea_body(a_ref, w_ref, b_ref, o_ref):
    o_ref[...] = jnp.dot(a_ref[...], w_ref[...],
                         preferred_element_type=jnp.float32) + b_ref[...]


def _prep_ea(ea_pad, w1a_e, b1a_row):
    grid = _EP // _BE
    return pl.pallas_call(
        _ea_body,
        grid=(grid,),
        in_specs=[
            pl.BlockSpec((_BE, _DE), lambda i: (i, 0)),
            pl.BlockSpec((_DE, _H), lambda i: (0, 0)),
            pl.BlockSpec((1, _H), lambda i: (0, 0)),
        ],
        out_specs=pl.BlockSpec((_BE, _H), lambda i: (i, 0)),
        out_shape=jax.ShapeDtypeStruct((_EP, _H), jnp.float32),
    )(ea_pad, w1a_e, b1a_row)


def _sc_edge_body(xa_hbm, ea_hbm, row_hbm, col_hbm, out_hbm,
                  shared, row_v, col_v, gath_v, ea_v, pay_v, sem):
    c = lax.axis_index("c")
    s = lax.axis_index("s")
    wid = s * _NC + c

    def _zero_row(r, _):
        for j in range(_WIDE // 16):
            pay_v[r, pl.ds(j * 16, 16)] = jnp.zeros((16,), jnp.float32)
        return _
    lax.fori_loop(0, _CHUNK, _zero_row, None)
    for k in range(_RPT // _CHUNK):
        pltpu.sync_copy(pay_v, shared.at[pl.ds(s * _RPT + k * _CHUNK, _CHUNK)])

    lane = lax.iota(jnp.int32, 16)
    cnt_vec = jnp.where(lane == 0, 1.0, 0.0).astype(jnp.float32)

    def _cnt_row(r, _):
        pay_v[r, pl.ds(_H, 16)] = cnt_vec
        return _
    lax.fori_loop(0, _CHUNK, _cnt_row, None)

    plsc.subcore_barrier()

    def _step(i, _):
        base = wid * _EPT + i * _CHUNK
        pltpu.sync_copy(row_hbm.at[pl.ds(base, _CHUNK)], row_v)
        pltpu.sync_copy(col_hbm.at[pl.ds(base, _CHUNK)], col_v)
        pltpu.sync_copy(ea_hbm.at[pl.ds(base, _CHUNK)], ea_v)
        pltpu.async_copy(xa_hbm.at[row_v], gath_v, sem).wait()

        def _row(r, __):
            for j in range(_H // 16):
                sl = pl.ds(j * 16, 16)
                pay_v[r, sl] = jnp.maximum(gath_v[r, sl] + ea_v[r, sl], 0.0)
            return __
        lax.fori_loop(0, _CHUNK, _row, None)

        pltpu.sync_copy(pay_v, shared.at[col_v], add=True)
        return _
    lax.fori_loop(0, _STEPS, _step, None)

    plsc.subcore_barrier()

    off = c * _NP + s * _RPT
    pltpu.sync_copy(shared.at[pl.ds(s * _RPT, _RPT)],
                    out_hbm.at[pl.ds(off, _RPT)])


def _sc_edge(xa, ea, row_pad, col_pad):
    mesh = plsc.VectorSubcoreMesh(core_axis_name="c", subcore_axis_name="s")
    kern = functools.partial(
        pl.kernel,
        out_type=jax.ShapeDtypeStruct((_NC * _NP, _WIDE), jnp.float32),
        mesh=mesh,
        scratch_types=[
            pltpu.VMEM_SHARED((_NP, _WIDE), jnp.float32),
            pltpu.VMEM((_CHUNK,), jnp.int32),
            pltpu.VMEM((_CHUNK,), jnp.int32),
            pltpu.VMEM((_CHUNK, _H), jnp.float32),
            pltpu.VMEM((_CHUNK, _H), jnp.float32),
            pltpu.VMEM((_CHUNK, _WIDE), jnp.float32),
            pltpu.SemaphoreType.DMA,
        ],
        compiler_params=pltpu.CompilerParams(use_tc_tiling_on_sc=False),
    )(_sc_edge_body)
    return kern(xa, ea, row_pad, col_pad)


_BN = 1024


def _node_body(x_ref, h0_ref, h1_ref, bt_ref, u_ref, w1b_ref, b1b_ref,
               w2ax_ref, w2am_ref, w2au_ref, b2a_ref, w2b_ref, b2b_ref,
               o_ref):
    h0 = h0_ref[...]
    h1 = h1_ref[...]
    hsum = h0[:, :_H] + h1[:, :_H]
    cnt = h0[:, _H:_H + 1] + h1[:, _H:_H + 1]
    inv = 1.0 / jnp.maximum(cnt, 1.0)
    meanh = hsum * inv
    ind = (cnt > 0.0).astype(jnp.float32)

    wc = jnp.dot(w1b_ref[...], w2am_ref[...],
                 preferred_element_type=jnp.float32)
    bc = jnp.dot(b1b_ref[...], w2am_ref[...],
                 preferred_element_type=jnp.float32)
    ub = jnp.dot(u_ref[...], w2au_ref[...],
                 preferred_element_type=jnp.float32)

    onehot = (bt_ref[...] == lax.broadcasted_iota(jnp.int32, (1, _B), 1)
              ).astype(jnp.float32)

    z = (jnp.dot(x_ref[...], w2ax_ref[...],
                 preferred_element_type=jnp.float32)
         + jnp.dot(meanh, wc, preferred_element_type=jnp.float32)
         + ind * bc
         + jnp.dot(onehot, ub, preferred_element_type=jnp.float32)
         + b2a_ref[...])
    o_ref[...] = jnp.dot(jnp.maximum(z, 0.0), w2b_ref[...],
                         preferred_element_type=jnp.float32) + b2b_ref[...]


def _node(x_pad, h0, h1, bt_col, u, w1b, b1b_row, w2a_x, w2a_m, w2a_u,
          b2a_row, w2b, b2b_row):
    grid = _NP // _BN
    full = lambda shape: pl.BlockSpec(shape, lambda i: tuple(0 for _ in shape))
    return pl.pallas_call(
        _node_body,
        grid=(grid,),
        in_specs=[
            pl.BlockSpec((_BN, _D), lambda i: (i, 0)),
            pl.BlockSpec((_BN, _WIDE), lambda i: (i, 0)),
            pl.BlockSpec((_BN, _WIDE), lambda i: (i, 0)),
            pl.BlockSpec((_BN, 1), lambda i: (i, 0)),
            full((_B, _DU)),
            full((_H, _H)),
            full((1, _H)),
            full((_D, _H)),
            full((_H, _H)),
            full((_DU, _H)),
            full((1, _H)),
            full((_H, _DOUT)),
            full((1, _DOUT)),
        ],
        out_specs=pl.BlockSpec((_BN, _DOUT), lambda i: (i, 0)),
        out_shape=jax.ShapeDtypeStruct((_NP, _DOUT), jnp.float32),
    )(x_pad, h0, h1, bt_col, u, w1b, b1b_row, w2a_x, w2a_m, w2a_u,
      b2a_row, w2b, b2b_row)


def kernel(x, edge_index, edge_attr, u, batch,
           W1a, b1a, W1b, b1b, W2a, b2a, W2b, b2b):
    x_pad = jnp.pad(x, ((0, _NP - _N), (0, 0)))
    row_pad = jnp.pad(edge_index[0], (0, _EP - _E))
    col_pad = jnp.pad(edge_index[1], (0, _EP - _E), constant_values=_N)
    ea_pad = jnp.pad(edge_attr, ((0, _EP - _E), (0, 0)))
    bt_col = jnp.pad(batch, (0, _NP - _N)).reshape(_NP, 1)

    xa = _prep_xa(x_pad, W1a[:_D])
    ea = _prep_ea(ea_pad, W1a[_D:], b1a.reshape(1, _H))
    hp = _sc_edge(xa, ea, row_pad, col_pad)

    out_pad = _node(x_pad, hp[:_NP], hp[_NP:], bt_col, u,
                    W1b, b1b.reshape(1, _H),
                    W2a[:_D], W2a[_D:_D + _H], W2a[_D + _H:],
                    b2a.reshape(1, _H), W2b, b2b.reshape(1, _DOUT))
    return out_pad[:_N]


# speedup vs baseline: 2.9848x; 1.5334x over previous
import functools

import jax
import jax.numpy as jnp
from jax import lax
from jax.experimental import pallas as pl
from jax.experimental.pallas import tpu as pltpu
from jax.experimental.pallas import tpu_sc as plsc

_N = 10000
_E = 320000
_D = 128
_DE = 16
_DU = 64
_H = 64
_DOUT = 128
_B = 16

_NP = 10240
_EP = 327680

_NC = 2
_NS = 16
_NW = _NC * _NS
_CHUNK = 128
_EPT = _EP // _NW
_STEPS = _EPT // _CHUNK
_WIDE = _H + 16
_RPT = _NP // _NS


def _xa_body(x_ref, w_ref, o_ref):
    o_ref[...] = jnp.dot(x_ref[...], w_ref[...],
                         preferred_element_type=jnp.float32)


def _prep_xa(x_pad, w1a_x):
    return pl.pallas_call(
        _xa_body,
        out_shape=jax.ShapeDtypeStruct((_NP, _H), jnp.float32),
    )(x_pad, w1a_x)


_BE = 8192


def _ea_body(a_ref, w_ref, b_ref, o_ref):
    o_ref[...] = jnp.dot(a_ref[...], w_ref[...],
                         preferred_element_type=jnp.float32) + b_ref[...]


def _prep_ea(ea_pad, w1a_e, b1a_row):
    grid = _EP // _BE
    return pl.pallas_call(
        _ea_body,
        grid=(grid,),
        in_specs=[
            pl.BlockSpec((_BE, _DE), lambda i: (i, 0)),
            pl.BlockSpec((_DE, _H), lambda i: (0, 0)),
            pl.BlockSpec((1, _H), lambda i: (0, 0)),
        ],
        out_specs=pl.BlockSpec((_BE, _H), lambda i: (i, 0)),
        out_shape=jax.ShapeDtypeStruct((_EP, _H), jnp.float32),
    )(ea_pad, w1a_e, b1a_row)


def _sc_edge_body(xa_hbm, ea_hbm, row_hbm, col_hbm, out_hbm,
                  shared, row0, row1, ea0, ea1, g0, g1, p0, p1,
                  c0, c1, c2, c3,
                  sin0, sin1, sg0, sg1,
                  sc0, sc1, sc2, sc3, ss0, ss1, ss2, ss3):
    c = lax.axis_index("c")
    s = lax.axis_index("s")
    wid = s * _NC + c
    ebase = wid * _EPT

    row_v = (row0, row1)
    ea_v = (ea0, ea1)
    gath_v = (g0, g1)
    pay_v = (p0, p1)
    col_v = (c0, c1, c2, c3)
    sem_in = (sin0, sin1)
    sem_g = (sg0, sg1)
    sem_c = (sc0, sc1, sc2, sc3)
    sem_s = (ss0, ss1, ss2, ss3)

    def _zero_row(r, _):
        for j in range(_WIDE // 16):
            p0[r, pl.ds(j * 16, 16)] = jnp.zeros((16,), jnp.float32)
        return _
    lax.fori_loop(0, _CHUNK, _zero_row, None)
    for k in range(_RPT // _CHUNK):
        pltpu.sync_copy(p0, shared.at[pl.ds(s * _RPT + k * _CHUNK, _CHUNK)])

    lane = lax.iota(jnp.int32, 16)
    cnt_vec = jnp.where(lane == 0, 1.0, 0.0).astype(jnp.float32)

    def _cnt_row(r, _):
        p0[r, pl.ds(_H, 16)] = cnt_vec
        p1[r, pl.ds(_H, 16)] = cnt_vec
        return _
    lax.fori_loop(0, _CHUNK, _cnt_row, None)

    plsc.subcore_barrier()

    def issue_in(i, b2, b4):
        base = ebase + i * _CHUNK
        pltpu.async_copy(row_hbm.at[pl.ds(base, _CHUNK)], row_v[b2],
                         sem_in[b2])
        pltpu.async_copy(ea_hbm.at[pl.ds(base, _CHUNK)], ea_v[b2],
                         sem_in[b2])
        pltpu.async_copy(col_hbm.at[pl.ds(base, _CHUNK)], col_v[b4],
                         sem_c[b4])

    def wait_in(b2):
        pltpu.make_async_copy(row_hbm.at[pl.ds(0, _CHUNK)], row_v[b2],
                              sem_in[b2]).wait()
        pltpu.make_async_copy(ea_hbm.at[pl.ds(0, _CHUNK)], ea_v[b2],
                              sem_in[b2]).wait()

    def wait_col(b4):
        pltpu.make_async_copy(col_hbm.at[pl.ds(0, _CHUNK)], col_v[b4],
                              sem_c[b4]).wait()

    def issue_gather(b2):
        pltpu.async_copy(xa_hbm.at[row_v[b2]], gath_v[b2], sem_g[b2])

    def wait_gather(b2):
        pltpu.make_async_copy(xa_hbm.at[row_v[b2]], gath_v[b2],
                              sem_g[b2]).wait()

    def issue_scatter(b2, b4):
        pltpu.async_copy(pay_v[b2], shared.at[col_v[b4]], sem_s[b4],
                         add=True)

    def wait_scatter(b2, b4):
        pltpu.make_async_copy(pay_v[b2], shared.at[col_v[b4]],
                              sem_s[b4]).wait()

    def compute(b2):
        gv, ev, pv = gath_v[b2], ea_v[b2], pay_v[b2]

        @plsc.parallel_loop(0, _CHUNK, 1, unroll=8)
        def _(r):
            for j in range(_H // 16):
                sl = pl.ds(j * 16, 16)
                pv[r, sl] = jnp.maximum(gv[r, sl] + ev[r, sl], 0.0)

    issue_in(0, 0, 0)
    wait_in(0)
    issue_gather(0)
    issue_in(1, 1, 1)

    def _pair(p, _):
        for q in range(4):
            i = 4 * p + q
            b2 = q & 1

            wait_gather(b2)

            @pl.when(i + 1 < _STEPS)
            def _():
                wait_in(1 - b2)
                issue_gather(1 - b2)

            @pl.when(i >= 2)
            def _():
                wait_scatter(b2, (q + 2) % 4)

            compute(b2)
            wait_col(q)
            issue_scatter(b2, q)

            @pl.when(i + 2 < _STEPS)
            def _():
                issue_in(i + 2, b2, (q + 2) % 4)
        return _
    lax.fori_loop(0, _STEPS // 4, _pair, None)

    wait_scatter(0, 2)
    wait_scatter(1, 3)

    plsc.subcore_barrier()

    off = c * _NP + s * _RPT
    pltpu.sync_copy(shared.at[pl.ds(s * _RPT, _RPT)],
                    out_hbm.at[pl.ds(off, _RPT)])


def _sc_edge(xa, ea, row_pad, col_pad):
    mesh = plsc.VectorSubcoreMesh(core_axis_name="c", subcore_axis_name="s")
    kern = functools.partial(
        pl.kernel,
        out_type=jax.ShapeDtypeStruct((_NC * _NP, _WIDE), jnp.float32),
        mesh=mesh,
        scratch_types=[
            pltpu.VMEM_SHARED((_NP, _WIDE), jnp.float32),
            pltpu.VMEM((_CHUNK,), jnp.int32),
            pltpu.VMEM((_CHUNK,), jnp.int32),
            pltpu.VMEM((_CHUNK, _H), jnp.float32),
            pltpu.VMEM((_CHUNK, _H), jnp.float32),
            pltpu.VMEM((_CHUNK, _H), jnp.float32),
            pltpu.VMEM((_CHUNK, _H), jnp.float32),
            pltpu.VMEM((_CHUNK, _WIDE), jnp.float32),
            pltpu.VMEM((_CHUNK, _WIDE), jnp.float32),
            pltpu.VMEM((_CHUNK,), jnp.int32),
            pltpu.VMEM((_CHUNK,), jnp.int32),
            pltpu.VMEM((_CHUNK,), jnp.int32),
            pltpu.VMEM((_CHUNK,), jnp.int32),
        ] + [pltpu.SemaphoreType.DMA] * 12,
        compiler_params=pltpu.CompilerParams(use_tc_tiling_on_sc=False),
    )(_sc_edge_body)
    return kern(xa, ea, row_pad, col_pad)


_BN = 1024


def _node_body(x_ref, h0_ref, h1_ref, bt_ref, u_ref, w1b_ref, b1b_ref,
               w2ax_ref, w2am_ref, w2au_ref, b2a_ref, w2b_ref, b2b_ref,
               o_ref):
    h0 = h0_ref[...]
    h1 = h1_ref[...]
    hsum = h0[:, :_H] + h1[:, :_H]
    cnt = h0[:, _H:_H + 1] + h1[:, _H:_H + 1]
    inv = 1.0 / jnp.maximum(cnt, 1.0)
    meanh = hsum * inv
    ind = (cnt > 0.0).astype(jnp.float32)

    wc = jnp.dot(w1b_ref[...], w2am_ref[...],
                 preferred_element_type=jnp.float32)
    bc = jnp.dot(b1b_ref[...], w2am_ref[...],
                 preferred_element_type=jnp.float32)
    ub = jnp.dot(u_ref[...], w2au_ref[...],
                 preferred_element_type=jnp.float32)

    onehot = (bt_ref[...] == lax.broadcasted_iota(jnp.int32, (1, _B), 1)
              ).astype(jnp.float32)

    z = (jnp.dot(x_ref[...], w2ax_ref[...],
                 preferred_element_type=jnp.float32)
         + jnp.dot(meanh, wc, preferred_element_type=jnp.float32)
         + ind * bc
         + jnp.dot(onehot, ub, preferred_element_type=jnp.float32)
         + b2a_ref[...])
    o_ref[...] = jnp.dot(jnp.maximum(z, 0.0), w2b_ref[...],
                         preferred_element_type=jnp.float32) + b2b_ref[...]


def _node(x_pad, h0, h1, bt_col, u, w1b, b1b_row, w2a_x, w2a_m, w2a_u,
          b2a_row, w2b, b2b_row):
    grid = _NP // _BN
    full = lambda shape: pl.BlockSpec(shape, lambda i: tuple(0 for _ in shape))
    return pl.pallas_call(
        _node_body,
        grid=(grid,),
        in_specs=[
            pl.BlockSpec((_BN, _D), lambda i: (i, 0)),
            pl.BlockSpec((_BN, _WIDE), lambda i: (i, 0)),
            pl.BlockSpec((_BN, _WIDE), lambda i: (i, 0)),
            pl.BlockSpec((_BN, 1), lambda i: (i, 0)),
            full((_B, _DU)),
            full((_H, _H)),
            full((1, _H)),
            full((_D, _H)),
            full((_H, _H)),
            full((_DU, _H)),
            full((1, _H)),
            full((_H, _DOUT)),
            full((1, _DOUT)),
        ],
        out_specs=pl.BlockSpec((_BN, _DOUT), lambda i: (i, 0)),
        out_shape=jax.ShapeDtypeStruct((_NP, _DOUT), jnp.float32),
    )(x_pad, h0, h1, bt_col, u, w1b, b1b_row, w2a_x, w2a_m, w2a_u,
      b2a_row, w2b, b2b_row)


def kernel(x, edge_index, edge_attr, u, batch,
           W1a, b1a, W1b, b1b, W2a, b2a, W2b, b2b):
    x_pad = jnp.pad(x, ((0, _NP - _N), (0, 0)))
    row_pad = jnp.pad(edge_index[0], (0, _EP - _E))
    col_pad = jnp.pad(edge_index[1], (0, _EP - _E), constant_values=_N)
    ea_pad = jnp.pad(edge_attr, ((0, _EP - _E), (0, 0)))
    bt_col = jnp.pad(batch, (0, _NP - _N)).reshape(_NP, 1)

    xa = _prep_xa(x_pad, W1a[:_D])
    ea = _prep_ea(ea_pad, W1a[_D:], b1a.reshape(1, _H))
    hp = _sc_edge(xa, ea, row_pad, col_pad)

    out_pad = _node(x_pad, hp[:_NP], hp[_NP:], bt_col, u,
                    W1b, b1b.reshape(1, _H),
                    W2a[:_D], W2a[_D:_D + _H], W2a[_D + _H:],
                    b2a.reshape(1, _H), W2b, b2b.reshape(1, _DOUT))
    return out_pad[:_N]


# speedup vs baseline: 3.9584x; 1.3262x over previous
import functools

import jax
import jax.numpy as jnp
from jax import lax
from jax.experimental import pallas as pl
from jax.experimental.pallas import tpu as pltpu
from jax.experimental.pallas import tpu_sc as plsc

_N = 10000
_E = 320000
_D = 128
_DE = 16
_DU = 64
_H = 64
_DOUT = 128
_B = 16

_NP = 10240
_EP = 327680

_NC = 2
_NS = 16
_NW = _NC * _NS
_CHUNK = 128
_EPT = _EP // _NW
_STEPS = _EPT // _CHUNK
_WIDE = _H + 16
_RPT = _NP // _NS


def _xa_body(x_ref, w_ref, o_ref):
    o_ref[...] = jnp.dot(x_ref[...], w_ref[...],
                         preferred_element_type=jnp.float32)


def _prep_xa(x_pad, w1a_x):
    return pl.pallas_call(
        _xa_body,
        out_shape=jax.ShapeDtypeStruct((_NP, _H), jnp.float32),
    )(x_pad, w1a_x)


_BE = 8192


def _ea_body(a_ref, w_ref, b_ref, o_ref):
    ea = jnp.dot(a_ref[...], w_ref[...],
                 preferred_element_type=jnp.float32) + b_ref[...]
    ea_r = ea.reshape(_BE // 128, 2, 64, _H)
    pair = jnp.concatenate([ea_r[:, 0], ea_r[:, 1]], axis=-1)
    o_ref[...] = pair.reshape(_BE // 2, 2 * _H)


def _prep_ea(edge_attr, w1a_e, b1a_row):
    grid = _EP // _BE
    return pl.pallas_call(
        _ea_body,
        grid=(grid,),
        in_specs=[
            pl.BlockSpec((_BE, _DE), lambda i: (i, 0)),
            pl.BlockSpec((_DE, _H), lambda i: (0, 0)),
            pl.BlockSpec((1, _H), lambda i: (0, 0)),
        ],
        out_specs=pl.BlockSpec((_BE // 2, 2 * _H), lambda i: (i, 0)),
        out_shape=jax.ShapeDtypeStruct((_EP // 2, 2 * _H), jnp.float32),
    )(edge_attr, w1a_e, b1a_row)


def _sc_edge_body(xa_hbm, ea_hbm, row_hbm, col_hbm, out_hbm,
                  shared, row0, row1, ea0, ea1, g0, g1, p0, p1,
                  c0, c1, c2, c3,
                  sin0, sin1, sg0, sg1,
                  sc0, sc1, sc2, sc3, ss0, ss1, ss2, ss3):
    c = lax.axis_index("c")
    s = lax.axis_index("s")
    wid = s * _NC + c
    ebase = wid * _EPT

    row_v = (row0, row1)
    ea_v = (ea0, ea1)
    gath_v = (g0, g1)
    pay_v = (p0, p1)
    col_v = (c0, c1, c2, c3)
    sem_in = (sin0, sin1)
    sem_g = (sg0, sg1)
    sem_c = (sc0, sc1, sc2, sc3)
    sem_s = (ss0, ss1, ss2, ss3)

    def _zero_row(r, _):
        for j in range(_WIDE // 16):
            p0[r, pl.ds(j * 16, 16)] = jnp.zeros((16,), jnp.float32)
        return _
    lax.fori_loop(0, _CHUNK, _zero_row, None)
    for k in range(_RPT // _CHUNK):
        pltpu.sync_copy(p0, shared.at[pl.ds(s * _RPT + k * _CHUNK, _CHUNK)])

    lane = lax.iota(jnp.int32, 16)
    cnt_vec = jnp.where(lane == 0, 1.0, 0.0).astype(jnp.float32)

    def _cnt_row(r, _):
        p0[r, pl.ds(_H, 16)] = cnt_vec
        p1[r, pl.ds(_H, 16)] = cnt_vec
        return _
    lax.fori_loop(0, _CHUNK, _cnt_row, None)

    plsc.subcore_barrier()

    ebase2 = wid * (_EPT // 2)

    def issue_in(i, b2, b4):
        base = ebase + i * _CHUNK
        base2 = ebase2 + i * (_CHUNK // 2)
        pltpu.async_copy(row_hbm.at[pl.ds(base, _CHUNK)], row_v[b2],
                         sem_in[b2])
        pltpu.async_copy(ea_hbm.at[pl.ds(base2, _CHUNK // 2)], ea_v[b2],
                         sem_in[b2])
        pltpu.async_copy(col_hbm.at[pl.ds(base, _CHUNK)], col_v[b4],
                         sem_c[b4])

    def wait_in(b2):
        pltpu.make_async_copy(row_hbm.at[pl.ds(0, _CHUNK)], row_v[b2],
                              sem_in[b2]).wait()
        pltpu.make_async_copy(ea_hbm.at[pl.ds(0, _CHUNK // 2)], ea_v[b2],
                              sem_in[b2]).wait()

    def wait_col(b4):
        pltpu.make_async_copy(col_hbm.at[pl.ds(0, _CHUNK)], col_v[b4],
                              sem_c[b4]).wait()

    def issue_gather(b2):
        pltpu.async_copy(xa_hbm.at[row_v[b2]], gath_v[b2], sem_g[b2])

    def wait_gather(b2):
        pltpu.make_async_copy(xa_hbm.at[row_v[b2]], gath_v[b2],
                              sem_g[b2]).wait()

    def issue_scatter(b2, b4):
        pltpu.async_copy(pay_v[b2], shared.at[col_v[b4]], sem_s[b4],
                         add=True)

    def wait_scatter(b2, b4):
        pltpu.make_async_copy(pay_v[b2], shared.at[col_v[b4]],
                              sem_s[b4]).wait()

    def compute(b2):
        gv, ev, pv = gath_v[b2], ea_v[b2], pay_v[b2]

        @plsc.parallel_loop(0, _CHUNK // 2, 1, unroll=4)
        def _(r2):
            for half in range(2):
                r = half * (_CHUNK // 2) + r2
                for j in range(_H // 16):
                    sl = pl.ds(j * 16, 16)
                    esl = pl.ds(half * _H + j * 16, 16)
                    pv[r, sl] = jnp.maximum(gv[r, sl] + ev[r2, esl], 0.0)

    issue_in(0, 0, 0)
    wait_in(0)
    issue_gather(0)
    issue_in(1, 1, 1)

    def _pair(p, _):
        for q in range(4):
            i = 4 * p + q
            b2 = q & 1

            wait_gather(b2)

            @pl.when(i + 1 < _STEPS)
            def _():
                wait_in(1 - b2)
                issue_gather(1 - b2)

            @pl.when(i >= 2)
            def _():
                wait_scatter(b2, (q + 2) % 4)

            compute(b2)
            wait_col(q)
            issue_scatter(b2, q)

            @pl.when(i + 2 < _STEPS)
            def _():
                issue_in(i + 2, b2, (q + 2) % 4)
        return _
    lax.fori_loop(0, _STEPS // 4, _pair, None)

    wait_scatter(0, 2)
    wait_scatter(1, 3)

    plsc.subcore_barrier()

    off = c * _NP + s * _RPT
    pltpu.sync_copy(shared.at[pl.ds(s * _RPT, _RPT)],
                    out_hbm.at[pl.ds(off, _RPT)])


def _sc_edge(xa, ea, row_pad, col_pad):
    mesh = plsc.VectorSubcoreMesh(core_axis_name="c", subcore_axis_name="s")
    kern = functools.partial(
        pl.kernel,
        out_type=jax.ShapeDtypeStruct((_NC * _NP, _WIDE), jnp.float32),
        mesh=mesh,
        scratch_types=[
            pltpu.VMEM_SHARED((_NP, _WIDE), jnp.float32),
            pltpu.VMEM((_CHUNK,), jnp.int32),
            pltpu.VMEM((_CHUNK,), jnp.int32),
            pltpu.VMEM((_CHUNK // 2, 2 * _H), jnp.float32),
            pltpu.VMEM((_CHUNK // 2, 2 * _H), jnp.float32),
            pltpu.VMEM((_CHUNK, _H), jnp.float32),
            pltpu.VMEM((_CHUNK, _H), jnp.float32),
            pltpu.VMEM((_CHUNK, _WIDE), jnp.float32),
            pltpu.VMEM((_CHUNK, _WIDE), jnp.float32),
            pltpu.VMEM((_CHUNK,), jnp.int32),
            pltpu.VMEM((_CHUNK,), jnp.int32),
            pltpu.VMEM((_CHUNK,), jnp.int32),
            pltpu.VMEM((_CHUNK,), jnp.int32),
        ] + [pltpu.SemaphoreType.DMA] * 12,
        compiler_params=pltpu.CompilerParams(use_tc_tiling_on_sc=False),
    )(_sc_edge_body)
    return kern(xa, ea, row_pad, col_pad)


_BN = 1024


def _node_body(x_ref, h0_ref, h1_ref, bt_ref, u_ref, w1b_ref, b1b_ref,
               w2ax_ref, w2am_ref, w2au_ref, b2a_ref, w2b_ref, b2b_ref,
               o_ref):
    h0 = h0_ref[...]
    h1 = h1_ref[...]
    hsum = h0[:, :_H] + h1[:, :_H]
    cnt = h0[:, _H:_H + 1] + h1[:, _H:_H + 1]
    inv = 1.0 / jnp.maximum(cnt, 1.0)
    meanh = hsum * inv
    ind = (cnt > 0.0).astype(jnp.float32)

    wc = jnp.dot(w1b_ref[...], w2am_ref[...],
                 preferred_element_type=jnp.float32)
    bc = jnp.dot(b1b_ref[...], w2am_ref[...],
                 preferred_element_type=jnp.float32)
    ub = jnp.dot(u_ref[...], w2au_ref[...],
                 preferred_element_type=jnp.float32)

    onehot = (bt_ref[...] == lax.broadcasted_iota(jnp.int32, (1, _B), 1)
              ).astype(jnp.float32)

    z = (jnp.dot(x_ref[...], w2ax_ref[...],
                 preferred_element_type=jnp.float32)
         + jnp.dot(meanh, wc, preferred_element_type=jnp.float32)
         + ind * bc
         + jnp.dot(onehot, ub, preferred_element_type=jnp.float32)
         + b2a_ref[...])
    o_ref[...] = jnp.dot(jnp.maximum(z, 0.0), w2b_ref[...],
                         preferred_element_type=jnp.float32) + b2b_ref[...]


def _node(x_pad, h0, h1, bt_col, u, w1b, b1b_row, w2a_x, w2a_m, w2a_u,
          b2a_row, w2b, b2b_row):
    grid = _NP // _BN
    full = lambda shape: pl.BlockSpec(shape, lambda i: tuple(0 for _ in shape))
    return pl.pallas_call(
        _node_body,
        grid=(grid,),
        in_specs=[
            pl.BlockSpec((_BN, _D), lambda i: (i, 0)),
            pl.BlockSpec((_BN, _WIDE), lambda i: (i, 0)),
            pl.BlockSpec((_BN, _WIDE), lambda i: (i, 0)),
            pl.BlockSpec((_BN, 1), lambda i: (i, 0)),
            full((_B, _DU)),
            full((_H, _H)),
            full((1, _H)),
            full((_D, _H)),
            full((_H, _H)),
            full((_DU, _H)),
            full((1, _H)),
            full((_H, _DOUT)),
            full((1, _DOUT)),
        ],
        out_specs=pl.BlockSpec((_BN, _DOUT), lambda i: (i, 0)),
        out_shape=jax.ShapeDtypeStruct((_NP, _DOUT), jnp.float32),
    )(x_pad, h0, h1, bt_col, u, w1b, b1b_row, w2a_x, w2a_m, w2a_u,
      b2a_row, w2b, b2b_row)


def kernel(x, edge_index, edge_attr, u, batch,
           W1a, b1a, W1b, b1b, W2a, b2a, W2b, b2b):
    x_pad = jnp.pad(x, ((0, _NP - _N), (0, 0)))
    row_pad = jnp.pad(edge_index[0], (0, _EP - _E))
    col_pad = jnp.pad(edge_index[1], (0, _EP - _E), constant_values=_N)
    bt_col = jnp.pad(batch, (0, _NP - _N)).reshape(_NP, 1)

    xa = _prep_xa(x_pad, W1a[:_D])
    ea = _prep_ea(edge_attr, W1a[_D:], b1a.reshape(1, _H))
    hp = _sc_edge(xa, ea, row_pad, col_pad)

    out_pad = _node(x_pad, hp[:_NP], hp[_NP:], bt_col, u,
                    W1b, b1b.reshape(1, _H),
                    W2a[:_D], W2a[_D:_D + _H], W2a[_D + _H:],
                    b2a.reshape(1, _H), W2b, b2b.reshape(1, _DOUT))
    return out_pad[:_N]


# speedup vs baseline: 4.1473x; 1.0477x over previous
import functools

import jax
import jax.numpy as jnp
from jax import lax
from jax.experimental import pallas as pl
from jax.experimental.pallas import tpu as pltpu
from jax.experimental.pallas import tpu_sc as plsc

_N = 10000
_E = 320000
_D = 128
_DE = 16
_DU = 64
_H = 64
_DOUT = 128
_B = 16

_NP = 10240
_EP = 327680

_NC = 2
_NS = 16
_NW = _NC * _NS
_CHUNK = 128
_EPT = _EP // _NW
_STEPS = _EPT // _CHUNK
_WIDE = _H + 16
_RPT = _NP // _NS


def _xa_body(x_ref, w_ref, o_ref):
    o_ref[...] = jnp.dot(x_ref[...], w_ref[...],
                         preferred_element_type=jnp.float32)


def _prep_xa(x_pad, w1a_x):
    return pl.pallas_call(
        _xa_body,
        out_shape=jax.ShapeDtypeStruct((_NP, _H), jnp.float32),
    )(x_pad, w1a_x)


_BE = 8192
_BR = _BE // 8


def _ea_body(a_ref, w_ref, b_ref, o_ref):
    ea8 = jnp.dot(a_ref[...], w_ref[...],
                  preferred_element_type=jnp.float32) + b_ref[...]
    ea8r = ea8.reshape(_BE // 128, 2, 8, 8 * _H)
    for a in range(2):
        for kb in range(4):
            o_ref[0, :, a, kb, :, :] = ea8r[:, a, :,
                                            kb * 2 * _H:(kb + 1) * 2 * _H]


def _prep_ea(attr128, w_bd, b_bd):
    grid = _EP // _BE
    return pl.pallas_call(
        _ea_body,
        grid=(grid,),
        in_specs=[
            pl.BlockSpec((_BR, 128), lambda i: (i, 0)),
            pl.BlockSpec((128, 8 * _H), lambda i: (0, 0)),
            pl.BlockSpec((1, 8 * _H), lambda i: (0, 0)),
        ],
        out_specs=pl.BlockSpec((1, _BE // 128, 2, 4, 8, 2 * _H),
                               lambda i: (i, 0, 0, 0, 0, 0)),
        out_shape=jax.ShapeDtypeStruct(
            (grid, _BE // 128, 2, 4, 8, 2 * _H), jnp.float32),
    )(attr128, w_bd, b_bd)


def _sc_edge_body(xa_hbm, ea_hbm, row_hbm, col_hbm, out_hbm,
                  shared, row0, row1, row2, row3, ea0, ea1,
                  g0, g1, g2, g3, p0, p1,
                  c0, c1, c2, c3,
                  sr0, sr1, sr2, sr3, se0, se1,
                  sg0, sg1, sg2, sg3,
                  sc0, sc1, sc2, sc3, ss0, ss1, ss2, ss3):
    c = lax.axis_index("c")
    s = lax.axis_index("s")
    wid = s * _NC + c
    ebase = wid * _EPT

    row_v = (row0, row1, row2, row3)
    ea_v = (ea0, ea1)
    gath_v = (g0, g1, g2, g3)
    pay_v = (p0, p1)
    col_v = (c0, c1, c2, c3)
    sem_r = (sr0, sr1, sr2, sr3)
    sem_e = (se0, se1)
    sem_g = (sg0, sg1, sg2, sg3)
    sem_c = (sc0, sc1, sc2, sc3)
    sem_s = (ss0, ss1, ss2, ss3)

    def _zero_row(r, _):
        for j in range(_WIDE // 16):
            p0[r, pl.ds(j * 16, 16)] = jnp.zeros((16,), jnp.float32)
        return _
    lax.fori_loop(0, _CHUNK, _zero_row, None)
    for k in range(_RPT // _CHUNK):
        pltpu.sync_copy(p0, shared.at[pl.ds(s * _RPT + k * _CHUNK, _CHUNK)])

    lane = lax.iota(jnp.int32, 16)
    cnt_vec = jnp.where(lane == 0, 1.0, 0.0).astype(jnp.float32)

    def _cnt_row(r, _):
        p0[r, pl.ds(_H, 16)] = cnt_vec
        p1[r, pl.ds(_H, 16)] = cnt_vec
        return _
    lax.fori_loop(0, _CHUNK, _cnt_row, None)

    plsc.subcore_barrier()

    ebase2 = wid * (_EPT // 2)

    def issue_row(i, r4):
        base = ebase + i * _CHUNK
        pltpu.async_copy(row_hbm.at[pl.ds(base, _CHUNK)], row_v[r4],
                         sem_r[r4])

    def issue_ea(i, b2):
        base2 = ebase2 + i * (_CHUNK // 2)
        pltpu.async_copy(ea_hbm.at[pl.ds(base2, _CHUNK // 2)], ea_v[b2],
                         sem_e[b2])

    def issue_col(i, c4):
        base = ebase + i * _CHUNK
        pltpu.async_copy(col_hbm.at[pl.ds(base, _CHUNK)], col_v[c4],
                         sem_c[c4])

    def wait_row(r4):
        pltpu.make_async_copy(row_hbm.at[pl.ds(0, _CHUNK)], row_v[r4],
                              sem_r[r4]).wait()

    def wait_ea(b2):
        pltpu.make_async_copy(ea_hbm.at[pl.ds(0, _CHUNK // 2)], ea_v[b2],
                              sem_e[b2]).wait()

    def wait_col(c4):
        pltpu.make_async_copy(col_hbm.at[pl.ds(0, _CHUNK)], col_v[c4],
                              sem_c[c4]).wait()

    def issue_gather(r4):
        pltpu.async_copy(xa_hbm.at[row_v[r4]], gath_v[r4], sem_g[r4])

    def wait_gather(r4):
        pltpu.make_async_copy(xa_hbm.at[row_v[r4]], gath_v[r4],
                              sem_g[r4]).wait()

    def issue_scatter(b2, c4):
        pltpu.async_copy(pay_v[b2], shared.at[col_v[c4]], sem_s[c4],
                         add=True)

    def wait_scatter(b2, c4):
        pltpu.make_async_copy(pay_v[b2], shared.at[col_v[c4]],
                              sem_s[c4]).wait()

    def compute(r4, b2):
        gv, ev, pv = gath_v[r4], ea_v[b2], pay_v[b2]

        @plsc.parallel_loop(0, _CHUNK // 2, 1, unroll=4)
        def _(lr):
            a = lr // 32
            kb = (lr % 32) // 8
            gb = lr % 8
            p0 = a * 64 + gb * 8 + kb * 2
            for half in range(2):
                r = p0 + half
                for j in range(_H // 16):
                    sl = pl.ds(j * 16, 16)
                    esl = pl.ds(half * _H + j * 16, 16)
                    pv[r, sl] = jnp.maximum(gv[r, sl] + ev[lr, esl], 0.0)

    issue_row(0, 0)
    issue_row(1, 1)
    issue_row(2, 2)
    issue_ea(0, 0)
    issue_ea(1, 1)
    issue_col(0, 0)
    issue_col(1, 1)
    wait_row(0)
    issue_gather(0)
    wait_row(1)
    issue_gather(1)

    def _quad(p, _):
        for q in range(4):
            i = 4 * p + q
            b2 = q & 1
            q2 = (q + 2) % 4
            q3 = (q + 3) % 4

            wait_gather(q)

            @pl.when(i + 2 < _STEPS)
            def _():
                wait_row(q2)
                issue_gather(q2)

            @pl.when(i >= 2)
            def _():
                wait_scatter(b2, q2)

            wait_ea(b2)
            compute(q, b2)
            wait_col(q)
            issue_scatter(b2, q)

            @pl.when(i + 3 < _STEPS)
            def _():
                issue_row(i + 3, q3)

            @pl.when(i + 2 < _STEPS)
            def _():
                issue_ea(i + 2, b2)
                issue_col(i + 2, q2)
        return _
    lax.fori_loop(0, _STEPS // 4, _quad, None)

    wait_scatter(0, 2)
    wait_scatter(1, 3)

    plsc.subcore_barrier()

    off = c * _NP + s * _RPT
    pltpu.sync_copy(shared.at[pl.ds(s * _RPT, _RPT)],
                    out_hbm.at[pl.ds(off, _RPT)])


def _sc_edge(xa, ea, row_pad, col_pad):
    mesh = plsc.VectorSubcoreMesh(core_axis_name="c", subcore_axis_name="s")
    kern = functools.partial(
        pl.kernel,
        out_type=jax.ShapeDtypeStruct((_NC * _NP, _WIDE), jnp.float32),
        mesh=mesh,
        scratch_types=(
            [pltpu.VMEM_SHARED((_NP, _WIDE), jnp.float32)]
            + [pltpu.VMEM((_CHUNK,), jnp.int32)] * 4
            + [pltpu.VMEM((_CHUNK // 2, 2 * _H), jnp.float32)] * 2
            + [pltpu.VMEM((_CHUNK, _H), jnp.float32)] * 4
            + [pltpu.VMEM((_CHUNK, _WIDE), jnp.float32)] * 2
            + [pltpu.VMEM((_CHUNK,), jnp.int32)] * 4
            + [pltpu.SemaphoreType.DMA] * 18
        ),
        compiler_params=pltpu.CompilerParams(use_tc_tiling_on_sc=False),
    )(_sc_edge_body)
    return kern(xa, ea, row_pad, col_pad)


_BN = 1024


def _node_body(x_ref, h0_ref, h1_ref, bt_ref, u_ref, w1b_ref, b1b_ref,
               w2ax_ref, w2am_ref, w2au_ref, b2a_ref, w2b_ref, b2b_ref,
               o_ref):
    h0 = h0_ref[...]
    h1 = h1_ref[...]
    hsum = h0[:, :_H] + h1[:, :_H]
    cnt = h0[:, _H:_H + 1] + h1[:, _H:_H + 1]
    inv = 1.0 / jnp.maximum(cnt, 1.0)
    meanh = hsum * inv
    ind = (cnt > 0.0).astype(jnp.float32)

    wc = jnp.dot(w1b_ref[...], w2am_ref[...],
                 preferred_element_type=jnp.float32)
    bc = jnp.dot(b1b_ref[...], w2am_ref[...],
                 preferred_element_type=jnp.float32)
    ub = jnp.dot(u_ref[...], w2au_ref[...],
                 preferred_element_type=jnp.float32)

    onehot = (bt_ref[...] == lax.broadcasted_iota(jnp.int32, (1, _B), 1)
              ).astype(jnp.float32)

    z = (jnp.dot(x_ref[...], w2ax_ref[...],
                 preferred_element_type=jnp.float32)
         + jnp.dot(meanh, wc, preferred_element_type=jnp.float32)
         + ind * bc
         + jnp.dot(onehot, ub, preferred_element_type=jnp.float32)
         + b2a_ref[...])
    o_ref[...] = jnp.dot(jnp.maximum(z, 0.0), w2b_ref[...],
                         preferred_element_type=jnp.float32) + b2b_ref[...]


def _node(x_pad, h0, h1, bt_col, u, w1b, b1b_row, w2a_x, w2a_m, w2a_u,
          b2a_row, w2b, b2b_row):
    grid = _NP // _BN
    full = lambda shape: pl.BlockSpec(shape, lambda i: tuple(0 for _ in shape))
    return pl.pallas_call(
        _node_body,
        grid=(grid,),
        in_specs=[
            pl.BlockSpec((_BN, _D), lambda i: (i, 0)),
            pl.BlockSpec((_BN, _WIDE), lambda i: (i, 0)),
            pl.BlockSpec((_BN, _WIDE), lambda i: (i, 0)),
            pl.BlockSpec((_BN, 1), lambda i: (i, 0)),
            full((_B, _DU)),
            full((_H, _H)),
            full((1, _H)),
            full((_D, _H)),
            full((_H, _H)),
            full((_DU, _H)),
            full((1, _H)),
            full((_H, _DOUT)),
            full((1, _DOUT)),
        ],
        out_specs=pl.BlockSpec((_BN, _DOUT), lambda i: (i, 0)),
        out_shape=jax.ShapeDtypeStruct((_NP, _DOUT), jnp.float32),
    )(x_pad, h0, h1, bt_col, u, w1b, b1b_row, w2a_x, w2a_m, w2a_u,
      b2a_row, w2b, b2b_row)


def kernel(x, edge_index, edge_attr, u, batch,
           W1a, b1a, W1b, b1b, W2a, b2a, W2b, b2b):
    x_pad = jnp.pad(x, ((0, _NP - _N), (0, 0)))
    row_pad = jnp.pad(edge_index[0], (0, _EP - _E))
    col_pad = jnp.pad(edge_index[1], (0, _EP - _E), constant_values=_N)
    bt_col = jnp.pad(batch, (0, _NP - _N)).reshape(_NP, 1)

    xa = _prep_xa(x_pad, W1a[:_D])
    attr128 = edge_attr.reshape(_E // 8, 8 * _DE)
    w_bd = jnp.kron(jnp.eye(8, dtype=jnp.float32), W1a[_D:])
    b_bd = jnp.tile(b1a, 8).reshape(1, 8 * _H)
    ea6 = _prep_ea(attr128, w_bd, b_bd)
    ea = ea6.reshape(_EP // 2, 2 * _H)
    hp = _sc_edge(xa, ea, row_pad, col_pad)

    out_pad = _node(x_pad, hp[:_NP], hp[_NP:], bt_col, u,
                    W1b, b1b.reshape(1, _H),
                    W2a[:_D], W2a[_D:_D + _H], W2a[_D + _H:],
                    b2a.reshape(1, _H), W2b, b2b.reshape(1, _DOUT))
    return out_pad[:_N]


# speedup vs baseline: 4.5750x; 1.1031x over previous
import functools

import jax
import jax.numpy as jnp
from jax import lax
from jax.experimental import pallas as pl
from jax.experimental.pallas import tpu as pltpu
from jax.experimental.pallas import tpu_sc as plsc

_N = 10000
_E = 320000
_D = 128
_DE = 16
_DU = 64
_H = 64
_DOUT = 128
_B = 16

_NP = 10240
_EP = 327680

_NC = 2
_NS = 16
_NW = _NC * _NS
_CHUNK = 128
_EPT = _EP // _NW
_STEPS = _EPT // _CHUNK
_WIDE = _H + 16
_RPT = _NP // _NS


def _xa_body(x_ref, w_ref, o_ref):
    o_ref[...] = jnp.dot(x_ref[...], w_ref[...],
                         preferred_element_type=jnp.float32)


def _prep_xa(x_pad, w1a_x):
    return pl.pallas_call(
        _xa_body,
        out_shape=jax.ShapeDtypeStruct((_NP, _H), jnp.float32),
    )(x_pad, w1a_x)


_BE = 4096


def _ea_body(at_ref, w_ref, b_ref, o_ref):
    attr = jnp.transpose(at_ref[...])
    ea = jnp.dot(attr, w_ref[...],
                 preferred_element_type=jnp.float32) + b_ref[...]
    ea_r = ea.reshape(_BE // 128, 2, 64, _H)
    pair = jnp.concatenate([ea_r[:, 0], ea_r[:, 1]], axis=-1)
    o_ref[...] = pair.reshape(_BE // 2, 2 * _H)


def _prep_ea(attr_t, w1a_e, b1a_row):
    grid = _EP // _BE
    return pl.pallas_call(
        _ea_body,
        grid=(grid,),
        in_specs=[
            pl.BlockSpec((_DE, _BE), lambda i: (0, i)),
            pl.BlockSpec((_DE, _H), lambda i: (0, 0)),
            pl.BlockSpec((1, _H), lambda i: (0, 0)),
        ],
        out_specs=pl.BlockSpec((_BE // 2, 2 * _H), lambda i: (i, 0)),
        out_shape=jax.ShapeDtypeStruct((_EP // 2, 2 * _H), jnp.float32),
    )(attr_t, w1a_e, b1a_row)


def _sc_edge_body(xa_hbm, ea_hbm, row_hbm, col_hbm, out_hbm,
                  shared, row0, row1, row2, row3, ea0, ea1,
                  g0, g1, g2, g3, p0, p1,
                  c0, c1, c2, c3,
                  sr0, sr1, sr2, sr3, se0, se1,
                  sg0, sg1, sg2, sg3,
                  sc0, sc1, sc2, sc3, ss0, ss1, ss2, ss3):
    c = lax.axis_index("c")
    s = lax.axis_index("s")
    wid = s * _NC + c
    ebase = wid * _EPT

    row_v = (row0, row1, row2, row3)
    ea_v = (ea0, ea1)
    gath_v = (g0, g1, g2, g3)
    pay_v = (p0, p1)
    col_v = (c0, c1, c2, c3)
    sem_r = (sr0, sr1, sr2, sr3)
    sem_e = (se0, se1)
    sem_g = (sg0, sg1, sg2, sg3)
    sem_c = (sc0, sc1, sc2, sc3)
    sem_s = (ss0, ss1, ss2, ss3)

    def _zero_row(r, _):
        for j in range(_WIDE // 16):
            p0[r, pl.ds(j * 16, 16)] = jnp.zeros((16,), jnp.float32)
        return _
    lax.fori_loop(0, _CHUNK, _zero_row, None)
    for k in range(_RPT // _CHUNK):
        pltpu.sync_copy(p0, shared.at[pl.ds(s * _RPT + k * _CHUNK, _CHUNK)])

    lane = lax.iota(jnp.int32, 16)
    cnt_vec = jnp.where(lane == 0, 1.0, 0.0).astype(jnp.float32)

    def _cnt_row(r, _):
        p0[r, pl.ds(_H, 16)] = cnt_vec
        p1[r, pl.ds(_H, 16)] = cnt_vec
        return _
    lax.fori_loop(0, _CHUNK, _cnt_row, None)

    plsc.subcore_barrier()

    ebase2 = wid * (_EPT // 2)

    def issue_row(i, r4):
        base = ebase + i * _CHUNK
        pltpu.async_copy(row_hbm.at[pl.ds(base, _CHUNK)], row_v[r4],
                         sem_r[r4])

    def issue_ea(i, b2):
        base2 = ebase2 + i * (_CHUNK // 2)
        pltpu.async_copy(ea_hbm.at[pl.ds(base2, _CHUNK // 2)], ea_v[b2],
                         sem_e[b2])

    def issue_col(i, c4):
        base = ebase + i * _CHUNK
        pltpu.async_copy(col_hbm.at[pl.ds(base, _CHUNK)], col_v[c4],
                         sem_c[c4])

    def wait_row(r4):
        pltpu.make_async_copy(row_hbm.at[pl.ds(0, _CHUNK)], row_v[r4],
                              sem_r[r4]).wait()

    def wait_ea(b2):
        pltpu.make_async_copy(ea_hbm.at[pl.ds(0, _CHUNK // 2)], ea_v[b2],
                              sem_e[b2]).wait()

    def wait_col(c4):
        pltpu.make_async_copy(col_hbm.at[pl.ds(0, _CHUNK)], col_v[c4],
                              sem_c[c4]).wait()

    def issue_gather(r4):
        pltpu.async_copy(xa_hbm.at[row_v[r4]], gath_v[r4], sem_g[r4])

    def wait_gather(r4):
        pltpu.make_async_copy(xa_hbm.at[row_v[r4]], gath_v[r4],
                              sem_g[r4]).wait()

    def issue_scatter(b2, c4):
        pltpu.async_copy(pay_v[b2], shared.at[col_v[c4]], sem_s[c4],
                         add=True)

    def wait_scatter(b2, c4):
        pltpu.make_async_copy(pay_v[b2], shared.at[col_v[c4]],
                              sem_s[c4]).wait()

    def compute(r4, b2):
        gv, ev, pv = gath_v[r4], ea_v[b2], pay_v[b2]

        @plsc.parallel_loop(0, _CHUNK // 2, 1, unroll=4)
        def _(r2):
            for half in range(2):
                r = half * (_CHUNK // 2) + r2
                for j in range(_H // 16):
                    sl = pl.ds(j * 16, 16)
                    esl = pl.ds(half * _H + j * 16, 16)
                    pv[r, sl] = jnp.maximum(gv[r, sl] + ev[r2, esl], 0.0)

    issue_row(0, 0)
    issue_row(1, 1)
    issue_row(2, 2)
    issue_ea(0, 0)
    issue_ea(1, 1)
    issue_col(0, 0)
    issue_col(1, 1)
    wait_row(0)
    issue_gather(0)
    wait_row(1)
    issue_gather(1)

    def _quad(p, _):
        for q in range(4):
            i = 4 * p + q
            b2 = q & 1
            q2 = (q + 2) % 4
            q3 = (q + 3) % 4

            wait_gather(q)

            @pl.when(i + 2 < _STEPS)
            def _():
                wait_row(q2)
                issue_gather(q2)

            @pl.when(i >= 2)
            def _():
                wait_scatter(b2, q2)

            wait_ea(b2)
            compute(q, b2)
            wait_col(q)
            issue_scatter(b2, q)

            @pl.when(i + 3 < _STEPS)
            def _():
                issue_row(i + 3, q3)

            @pl.when(i + 2 < _STEPS)
            def _():
                issue_ea(i + 2, b2)
                issue_col(i + 2, q2)
        return _
    lax.fori_loop(0, _STEPS // 4, _quad, None)

    wait_scatter(0, 2)
    wait_scatter(1, 3)

    plsc.subcore_barrier()

    off = c * _NP + s * _RPT
    pltpu.sync_copy(shared.at[pl.ds(s * _RPT, _RPT)],
                    out_hbm.at[pl.ds(off, _RPT)])


def _sc_edge(xa, ea, row_pad, col_pad):
    mesh = plsc.VectorSubcoreMesh(core_axis_name="c", subcore_axis_name="s")
    kern = functools.partial(
        pl.kernel,
        out_type=jax.ShapeDtypeStruct((_NC * _NP, _WIDE), jnp.float32),
        mesh=mesh,
        scratch_types=(
            [pltpu.VMEM_SHARED((_NP, _WIDE), jnp.float32)]
            + [pltpu.VMEM((_CHUNK,), jnp.int32)] * 4
            + [pltpu.VMEM((_CHUNK // 2, 2 * _H), jnp.float32)] * 2
            + [pltpu.VMEM((_CHUNK, _H), jnp.float32)] * 4
            + [pltpu.VMEM((_CHUNK, _WIDE), jnp.float32)] * 2
            + [pltpu.VMEM((_CHUNK,), jnp.int32)] * 4
            + [pltpu.SemaphoreType.DMA] * 18
        ),
        compiler_params=pltpu.CompilerParams(use_tc_tiling_on_sc=False),
    )(_sc_edge_body)
    return kern(xa, ea, row_pad, col_pad)


_BN = 1024


def _node_body(x_ref, h0_ref, h1_ref, bt_ref, u_ref, w1b_ref, b1b_ref,
               w2ax_ref, w2am_ref, w2au_ref, b2a_ref, w2b_ref, b2b_ref,
               o_ref):
    h0 = h0_ref[...]
    h1 = h1_ref[...]
    hsum = h0[:, :_H] + h1[:, :_H]
    cnt = h0[:, _H:_H + 1] + h1[:, _H:_H + 1]
    inv = 1.0 / jnp.maximum(cnt, 1.0)
    meanh = hsum * inv
    ind = (cnt > 0.0).astype(jnp.float32)

    wc = jnp.dot(w1b_ref[...], w2am_ref[...],
                 preferred_element_type=jnp.float32)
    bc = jnp.dot(b1b_ref[...], w2am_ref[...],
                 preferred_element_type=jnp.float32)
    ub = jnp.dot(u_ref[...], w2au_ref[...],
                 preferred_element_type=jnp.float32)

    onehot = (bt_ref[...] == lax.broadcasted_iota(jnp.int32, (1, _B), 1)
              ).astype(jnp.float32)

    z = (jnp.dot(x_ref[...], w2ax_ref[...],
                 preferred_element_type=jnp.float32)
         + jnp.dot(meanh, wc, preferred_element_type=jnp.float32)
         + ind * bc
         + jnp.dot(onehot, ub, preferred_element_type=jnp.float32)
         + b2a_ref[...])
    o_ref[...] = jnp.dot(jnp.maximum(z, 0.0), w2b_ref[...],
                         preferred_element_type=jnp.float32) + b2b_ref[...]


def _node(x_pad, h0, h1, bt_col, u, w1b, b1b_row, w2a_x, w2a_m, w2a_u,
          b2a_row, w2b, b2b_row):
    grid = _NP // _BN
    full = lambda shape: pl.BlockSpec(shape, lambda i: tuple(0 for _ in shape))
    return pl.pallas_call(
        _node_body,
        grid=(grid,),
        in_specs=[
            pl.BlockSpec((_BN, _D), lambda i: (i, 0)),
            pl.BlockSpec((_BN, _WIDE), lambda i: (i, 0)),
            pl.BlockSpec((_BN, _WIDE), lambda i: (i, 0)),
            pl.BlockSpec((_BN, 1), lambda i: (i, 0)),
            full((_B, _DU)),
            full((_H, _H)),
            full((1, _H)),
            full((_D, _H)),
            full((_H, _H)),
            full((_DU, _H)),
            full((1, _H)),
            full((_H, _DOUT)),
            full((1, _DOUT)),
        ],
        out_specs=pl.BlockSpec((_BN, _DOUT), lambda i: (i, 0)),
        out_shape=jax.ShapeDtypeStruct((_NP, _DOUT), jnp.float32),
    )(x_pad, h0, h1, bt_col, u, w1b, b1b_row, w2a_x, w2a_m, w2a_u,
      b2a_row, w2b, b2b_row)


def kernel(x, edge_index, edge_attr, u, batch,
           W1a, b1a, W1b, b1b, W2a, b2a, W2b, b2b):
    x_pad = jnp.pad(x, ((0, _NP - _N), (0, 0)))
    row_pad = jnp.pad(edge_index[0], (0, _EP - _E))
    col_pad = jnp.pad(edge_index[1], (0, _EP - _E), constant_values=_N)
    bt_col = jnp.pad(batch, (0, _NP - _N)).reshape(_NP, 1)

    xa = _prep_xa(x_pad, W1a[:_D])
    attr_t = jnp.pad(edge_attr.T, ((0, 0), (0, _EP - _E)))
    ea = _prep_ea(attr_t, W1a[_D:], b1a.reshape(1, _H))
    hp = _sc_edge(xa, ea, row_pad, col_pad)

    out_pad = _node(x_pad, hp[:_NP], hp[_NP:], bt_col, u,
                    W1b, b1b.reshape(1, _H),
                    W2a[:_D], W2a[_D:_D + _H], W2a[_D + _H:],
                    b2a.reshape(1, _H), W2b, b2b.reshape(1, _DOUT))
    return out_pad[:_N]


# speedup vs baseline: 4.8174x; 1.0530x over previous
import functools

import jax
import jax.numpy as jnp
from jax import lax
from jax.experimental import pallas as pl
from jax.experimental.pallas import tpu as pltpu
from jax.experimental.pallas import tpu_sc as plsc

_N = 10000
_E = 320000
_D = 128
_DE = 16
_DU = 64
_H = 64
_DOUT = 128
_B = 16

_NP = 10240
_EP = 327680

_NC = 2
_NS = 16
_NW = _NC * _NS
_CHUNK = 128
_EPT = _EP // _NW
_STEPS = _EPT // _CHUNK
_STEPS0 = 112
_STEPS1 = 48
_WIDE = _H + 16
_RPT = _NP // _NS


def _xa_body(x_ref, w_ref, o_ref):
    o_ref[...] = jnp.dot(x_ref[...], w_ref[...],
                         preferred_element_type=jnp.float32)


def _prep_xa(x_pad, w1a_x):
    return pl.pallas_call(
        _xa_body,
        out_shape=jax.ShapeDtypeStruct((_NP, _H), jnp.float32),
    )(x_pad, w1a_x)


_BE = 4096


def _ea_body(at_ref, w_ref, b_ref, o_ref):
    attr = jnp.transpose(at_ref[...])
    ea = jnp.dot(attr, w_ref[...],
                 preferred_element_type=jnp.float32) + b_ref[...]
    ea_r = ea.reshape(_BE // 128, 2, 64, _H)
    pair = jnp.concatenate([ea_r[:, 0], ea_r[:, 1]], axis=-1)
    o_ref[...] = pair.reshape(_BE // 2, 2 * _H)


def _prep_ea(attr_t, w1a_e, b1a_row):
    grid = _EP // _BE
    return pl.pallas_call(
        _ea_body,
        grid=(grid,),
        in_specs=[
            pl.BlockSpec((_DE, _BE), lambda i: (0, i)),
            pl.BlockSpec((_DE, _H), lambda i: (0, 0)),
            pl.BlockSpec((1, _H), lambda i: (0, 0)),
        ],
        out_specs=pl.BlockSpec((_BE // 2, 2 * _H), lambda i: (i, 0)),
        out_shape=jax.ShapeDtypeStruct((_EP // 2, 2 * _H), jnp.float32),
    )(attr_t, w1a_e, b1a_row)


def _sc_edge_body(xa_hbm, ea_hbm, row_hbm, col_hbm, out_hbm,
                  shared, row0, row1, row2, row3, ea0, ea1,
                  g0, g1, g2, g3, p0, p1,
                  c0, c1, c2, c3,
                  sr0, sr1, sr2, sr3, se0, se1,
                  sg0, sg1, sg2, sg3,
                  sc0, sc1, sc2, sc3, ss0, ss1, ss2, ss3):
    c = lax.axis_index("c")
    s = lax.axis_index("s")
    steps = jnp.where(c == 0, _STEPS0, _STEPS1)
    ebase = jnp.where(c == 0, s * (_STEPS0 * _CHUNK),
                      _NS * _STEPS0 * _CHUNK + s * (_STEPS1 * _CHUNK))

    row_v = (row0, row1, row2, row3)
    ea_v = (ea0, ea1)
    gath_v = (g0, g1, g2, g3)
    pay_v = (p0, p1)
    col_v = (c0, c1, c2, c3)
    sem_r = (sr0, sr1, sr2, sr3)
    sem_e = (se0, se1)
    sem_g = (sg0, sg1, sg2, sg3)
    sem_c = (sc0, sc1, sc2, sc3)
    sem_s = (ss0, ss1, ss2, ss3)

    def _zero_row(r, _):
        for j in range(_WIDE // 16):
            p0[r, pl.ds(j * 16, 16)] = jnp.zeros((16,), jnp.float32)
        return _
    lax.fori_loop(0, _CHUNK, _zero_row, None)
    for k in range(_RPT // _CHUNK):
        pltpu.sync_copy(p0, shared.at[pl.ds(s * _RPT + k * _CHUNK, _CHUNK)])

    lane = lax.iota(jnp.int32, 16)
    cnt_vec = jnp.where(lane == 0, 1.0, 0.0).astype(jnp.float32)

    def _cnt_row(r, _):
        p0[r, pl.ds(_H, 16)] = cnt_vec
        p1[r, pl.ds(_H, 16)] = cnt_vec
        return _
    lax.fori_loop(0, _CHUNK, _cnt_row, None)

    plsc.subcore_barrier()

    ebase2 = ebase // 2

    def issue_row(i, r4):
        base = ebase + i * _CHUNK
        pltpu.async_copy(row_hbm.at[pl.ds(base, _CHUNK)], row_v[r4],
                         sem_r[r4])

    def issue_ea(i, b2):
        base2 = ebase2 + i * (_CHUNK // 2)
        pltpu.async_copy(ea_hbm.at[pl.ds(base2, _CHUNK // 2)], ea_v[b2],
                         sem_e[b2])

    def issue_col(i, c4):
        base = ebase + i * _CHUNK
        pltpu.async_copy(col_hbm.at[pl.ds(base, _CHUNK)], col_v[c4],
                         sem_c[c4])

    def wait_row(r4):
        pltpu.make_async_copy(row_hbm.at[pl.ds(0, _CHUNK)], row_v[r4],
                              sem_r[r4]).wait()

    def wait_ea(b2):
        pltpu.make_async_copy(ea_hbm.at[pl.ds(0, _CHUNK // 2)], ea_v[b2],
                              sem_e[b2]).wait()

    def wait_col(c4):
        pltpu.make_async_copy(col_hbm.at[pl.ds(0, _CHUNK)], col_v[c4],
                              sem_c[c4]).wait()

    def issue_gather(r4):
        pltpu.async_copy(xa_hbm.at[row_v[r4]], gath_v[r4], sem_g[r4])

    def wait_gather(r4):
        pltpu.make_async_copy(xa_hbm.at[row_v[r4]], gath_v[r4],
                              sem_g[r4]).wait()

    def issue_scatter(b2, c4):
        pltpu.async_copy(pay_v[b2], shared.at[col_v[c4]], sem_s[c4],
                         add=True)

    def wait_scatter(b2, c4):
        pltpu.make_async_copy(pay_v[b2], shared.at[col_v[c4]],
                              sem_s[c4]).wait()

    def compute(r4, b2):
        gv, ev, pv = gath_v[r4], ea_v[b2], pay_v[b2]

        @plsc.parallel_loop(0, _CHUNK // 2, 1, unroll=4)
        def _(r2):
            for half in range(2):
                r = half * (_CHUNK // 2) + r2
                for j in range(_H // 16):
                    sl = pl.ds(j * 16, 16)
                    esl = pl.ds(half * _H + j * 16, 16)
                    pv[r, sl] = jnp.maximum(gv[r, sl] + ev[r2, esl], 0.0)

    issue_row(0, 0)
    issue_row(1, 1)
    issue_row(2, 2)
    issue_ea(0, 0)
    issue_ea(1, 1)
    issue_col(0, 0)
    issue_col(1, 1)
    wait_row(0)
    issue_gather(0)
    wait_row(1)
    issue_gather(1)

    def _quad(p, _):
        for q in range(4):
            i = 4 * p + q
            b2 = q & 1
            q2 = (q + 2) % 4
            q3 = (q + 3) % 4

            wait_gather(q)

            @pl.when(i + 2 < steps)
            def _():
                wait_row(q2)
                issue_gather(q2)

            @pl.when(i >= 2)
            def _():
                wait_scatter(b2, q2)

            wait_ea(b2)
            compute(q, b2)
            wait_col(q)
            issue_scatter(b2, q)

            @pl.when(i + 3 < steps)
            def _():
                issue_row(i + 3, q3)

            @pl.when(i + 2 < steps)
            def _():
                issue_ea(i + 2, b2)
                issue_col(i + 2, q2)
        return _
    lax.fori_loop(0, steps // 4, _quad, None)

    wait_scatter(0, 2)
    wait_scatter(1, 3)

    plsc.subcore_barrier()

    off = c * _NP + s * _RPT
    pltpu.sync_copy(shared.at[pl.ds(s * _RPT, _RPT)],
                    out_hbm.at[pl.ds(off, _RPT)])


def _sc_edge(xa, ea, row_pad, col_pad):
    mesh = plsc.VectorSubcoreMesh(core_axis_name="c", subcore_axis_name="s")
    kern = functools.partial(
        pl.kernel,
        out_type=jax.ShapeDtypeStruct((_NC * _NP, _WIDE), jnp.float32),
        mesh=mesh,
        scratch_types=(
            [pltpu.VMEM_SHARED((_NP, _WIDE), jnp.float32)]
            + [pltpu.VMEM((_CHUNK,), jnp.int32)] * 4
            + [pltpu.VMEM((_CHUNK // 2, 2 * _H), jnp.float32)] * 2
            + [pltpu.VMEM((_CHUNK, _H), jnp.float32)] * 4
            + [pltpu.VMEM((_CHUNK, _WIDE), jnp.float32)] * 2
            + [pltpu.VMEM((_CHUNK,), jnp.int32)] * 4
            + [pltpu.SemaphoreType.DMA] * 18
        ),
        compiler_params=pltpu.CompilerParams(use_tc_tiling_on_sc=False),
    )(_sc_edge_body)
    return kern(xa, ea, row_pad, col_pad)


_BN = 1024


def _node_body(x_ref, h0_ref, h1_ref, bt_ref, u_ref, w1b_ref, b1b_ref,
               w2ax_ref, w2am_ref, w2au_ref, b2a_ref, w2b_ref, b2b_ref,
               o_ref):
    h0 = h0_ref[...]
    h1 = h1_ref[...]
    hsum = h0[:, :_H] + h1[:, :_H]
    cnt = h0[:, _H:_H + 1] + h1[:, _H:_H + 1]
    inv = 1.0 / jnp.maximum(cnt, 1.0)
    meanh = hsum * inv
    ind = (cnt > 0.0).astype(jnp.float32)

    wc = jnp.dot(w1b_ref[...], w2am_ref[...],
                 preferred_element_type=jnp.float32)
    bc = jnp.dot(b1b_ref[...], w2am_ref[...],
                 preferred_element_type=jnp.float32)
    ub = jnp.dot(u_ref[...], w2au_ref[...],
                 preferred_element_type=jnp.float32)

    onehot = (bt_ref[...] == lax.broadcasted_iota(jnp.int32, (1, _B), 1)
              ).astype(jnp.float32)

    z = (jnp.dot(x_ref[...], w2ax_ref[...],
                 preferred_element_type=jnp.float32)
         + jnp.dot(meanh, wc, preferred_element_type=jnp.float32)
         + ind * bc
         + jnp.dot(onehot, ub, preferred_element_type=jnp.float32)
         + b2a_ref[...])
    o_ref[...] = jnp.dot(jnp.maximum(z, 0.0), w2b_ref[...],
                         preferred_element_type=jnp.float32) + b2b_ref[...]


def _node(x_pad, h0, h1, bt_col, u, w1b, b1b_row, w2a_x, w2a_m, w2a_u,
          b2a_row, w2b, b2b_row):
    grid = _NP // _BN
    full = lambda shape: pl.BlockSpec(shape, lambda i: tuple(0 for _ in shape))
    return pl.pallas_call(
        _node_body,
        grid=(grid,),
        in_specs=[
            pl.BlockSpec((_BN, _D), lambda i: (i, 0)),
            pl.BlockSpec((_BN, _WIDE), lambda i: (i, 0)),
            pl.BlockSpec((_BN, _WIDE), lambda i: (i, 0)),
            pl.BlockSpec((_BN, 1), lambda i: (i, 0)),
            full((_B, _DU)),
            full((_H, _H)),
            full((1, _H)),
            full((_D, _H)),
            full((_H, _H)),
            full((_DU, _H)),
            full((1, _H)),
            full((_H, _DOUT)),
            full((1, _DOUT)),
        ],
        out_specs=pl.BlockSpec((_BN, _DOUT), lambda i: (i, 0)),
        out_shape=jax.ShapeDtypeStruct((_NP, _DOUT), jnp.float32),
    )(x_pad, h0, h1, bt_col, u, w1b, b1b_row, w2a_x, w2a_m, w2a_u,
      b2a_row, w2b, b2b_row)


def kernel(x, edge_index, edge_attr, u, batch,
           W1a, b1a, W1b, b1b, W2a, b2a, W2b, b2b):
    x_pad = jnp.pad(x, ((0, _NP - _N), (0, 0)))
    row_pad = jnp.pad(edge_index[0], (0, _EP - _E))
    col_pad = jnp.pad(edge_index[1], (0, _EP - _E), constant_values=_N)
    bt_col = jnp.pad(batch, (0, _NP - _N)).reshape(_NP, 1)

    xa = _prep_xa(x_pad, W1a[:_D])
    attr_t = jnp.pad(edge_attr.T, ((0, 0), (0, _EP - _E)))
    ea = _prep_ea(attr_t, W1a[_D:], b1a.reshape(1, _H))
    hp = _sc_edge(xa, ea, row_pad, col_pad)

    out_pad = _node(x_pad, hp[:_NP], hp[_NP:], bt_col, u,
                    W1b, b1b.reshape(1, _H),
                    W2a[:_D], W2a[_D:_D + _H], W2a[_D + _H:],
                    b2a.reshape(1, _H), W2b, b2b.reshape(1, _DOUT))
    return out_pad[:_N]


# speedup vs baseline: 5.1319x; 1.0653x over previous
import functools

import jax
import jax.numpy as jnp
from jax import lax
from jax.experimental import pallas as pl
from jax.experimental.pallas import tpu as pltpu
from jax.experimental.pallas import tpu_sc as plsc

_N = 10000
_E = 320000
_D = 128
_DE = 16
_DU = 64
_H = 64
_DOUT = 128
_B = 16

_NP = 10240
_EP = 327680

_NC = 2
_NS = 16
_NW = _NC * _NS
_CHUNK = 128
_EPT = _EP // _NW
_STEPS = _EPT // _CHUNK
_STEPS0 = 144
_STEPS1 = 16
_WIDE = _H + 16
_RPT = _NP // _NS


def _xa_body(x_ref, w_ref, o_ref):
    o_ref[...] = jnp.dot(x_ref[...], w_ref[...],
                         preferred_element_type=jnp.float32)


def _prep_xa(x_pad, w1a_x):
    return pl.pallas_call(
        _xa_body,
        out_shape=jax.ShapeDtypeStruct((_NP, _H), jnp.float32),
    )(x_pad, w1a_x)


_BE = 4096


def _ea_body(at_ref, w_ref, b_ref, o_ref):
    attr = jnp.transpose(at_ref[...])
    ea = jnp.dot(attr, w_ref[...],
                 preferred_element_type=jnp.float32) + b_ref[...]
    ea_r = ea.reshape(_BE // 128, 2, 64, _H)
    pair = jnp.concatenate([ea_r[:, 0], ea_r[:, 1]], axis=-1)
    o_ref[...] = pair.reshape(_BE // 2, 2 * _H)


def _prep_ea(attr_t, w1a_e, b1a_row):
    grid = _EP // _BE
    return pl.pallas_call(
        _ea_body,
        grid=(grid,),
        in_specs=[
            pl.BlockSpec((_DE, _BE), lambda i: (0, i)),
            pl.BlockSpec((_DE, _H), lambda i: (0, 0)),
            pl.BlockSpec((1, _H), lambda i: (0, 0)),
        ],
        out_specs=pl.BlockSpec((_BE // 2, 2 * _H), lambda i: (i, 0)),
        out_shape=jax.ShapeDtypeStruct((_EP // 2, 2 * _H), jnp.float32),
    )(attr_t, w1a_e, b1a_row)


def _sc_edge_body(xa_hbm, ea_hbm, row_hbm, col_hbm, out_hbm,
                  shared, row0, row1, row2, row3, ea0, ea1,
                  g0, g1, g2, g3, p0, p1,
                  c0, c1, c2, c3,
                  sr0, sr1, sr2, sr3, se0, se1,
                  sg0, sg1, sg2, sg3,
                  sc0, sc1, sc2, sc3, ss0, ss1, ss2, ss3):
    c = lax.axis_index("c")
    s = lax.axis_index("s")
    steps = jnp.where(c == 0, _STEPS0, _STEPS1)
    ebase = jnp.where(c == 0, s * (_STEPS0 * _CHUNK),
                      _NS * _STEPS0 * _CHUNK + s * (_STEPS1 * _CHUNK))

    row_v = (row0, row1, row2, row3)
    ea_v = (ea0, ea1)
    gath_v = (g0, g1, g2, g3)
    pay_v = (p0, p1)
    col_v = (c0, c1, c2, c3)
    sem_r = (sr0, sr1, sr2, sr3)
    sem_e = (se0, se1)
    sem_g = (sg0, sg1, sg2, sg3)
    sem_c = (sc0, sc1, sc2, sc3)
    sem_s = (ss0, ss1, ss2, ss3)

    def _zero_row(r, _):
        for j in range(_WIDE // 16):
            p0[r, pl.ds(j * 16, 16)] = jnp.zeros((16,), jnp.float32)
        return _
    lax.fori_loop(0, _CHUNK, _zero_row, None)
    for k in range(_RPT // _CHUNK):
        pltpu.sync_copy(p0, shared.at[pl.ds(s * _RPT + k * _CHUNK, _CHUNK)])

    lane = lax.iota(jnp.int32, 16)
    cnt_vec = jnp.where(lane == 0, 1.0, 0.0).astype(jnp.float32)

    def _cnt_row(r, _):
        p0[r, pl.ds(_H, 16)] = cnt_vec
        p1[r, pl.ds(_H, 16)] = cnt_vec
        return _
    lax.fori_loop(0, _CHUNK, _cnt_row, None)

    plsc.subcore_barrier()

    ebase2 = ebase // 2

    def issue_row(i, r4):
        base = ebase + i * _CHUNK
        pltpu.async_copy(row_hbm.at[pl.ds(base, _CHUNK)], row_v[r4],
                         sem_r[r4])

    def issue_ea(i, b2):
        base2 = ebase2 + i * (_CHUNK // 2)
        pltpu.async_copy(ea_hbm.at[pl.ds(base2, _CHUNK // 2)], ea_v[b2],
                         sem_e[b2])

    def issue_col(i, c4):
        base = ebase + i * _CHUNK
        pltpu.async_copy(col_hbm.at[pl.ds(base, _CHUNK)], col_v[c4],
                         sem_c[c4])

    def wait_row(r4):
        pltpu.make_async_copy(row_hbm.at[pl.ds(0, _CHUNK)], row_v[r4],
                              sem_r[r4]).wait()

    def wait_ea(b2):
        pltpu.make_async_copy(ea_hbm.at[pl.ds(0, _CHUNK // 2)], ea_v[b2],
                              sem_e[b2]).wait()

    def wait_col(c4):
        pltpu.make_async_copy(col_hbm.at[pl.ds(0, _CHUNK)], col_v[c4],
                              sem_c[c4]).wait()

    def issue_gather(r4):
        pltpu.async_copy(xa_hbm.at[row_v[r4]], gath_v[r4], sem_g[r4])

    def wait_gather(r4):
        pltpu.make_async_copy(xa_hbm.at[row_v[r4]], gath_v[r4],
                              sem_g[r4]).wait()

    def issue_scatter(b2, c4):
        pltpu.async_copy(pay_v[b2], shared.at[col_v[c4]], sem_s[c4],
                         add=True)

    def wait_scatter(b2, c4):
        pltpu.make_async_copy(pay_v[b2], shared.at[col_v[c4]],
                              sem_s[c4]).wait()

    def compute(r4, b2):
        gv, ev, pv = gath_v[r4], ea_v[b2], pay_v[b2]

        @plsc.parallel_loop(0, _CHUNK // 2, 1, unroll=4)
        def _(r2):
            for half in range(2):
                r = half * (_CHUNK // 2) + r2
                for j in range(_H // 16):
                    sl = pl.ds(j * 16, 16)
                    esl = pl.ds(half * _H + j * 16, 16)
                    pv[r, sl] = jnp.maximum(gv[r, sl] + ev[r2, esl], 0.0)

    issue_row(0, 0)
    issue_row(1, 1)
    issue_row(2, 2)
    issue_ea(0, 0)
    issue_ea(1, 1)
    issue_col(0, 0)
    issue_col(1, 1)
    wait_row(0)
    issue_gather(0)
    wait_row(1)
    issue_gather(1)

    def _quad(p, _):
        for q in range(4):
            i = 4 * p + q
            b2 = q & 1
            q2 = (q + 2) % 4
            q3 = (q + 3) % 4

            wait_gather(q)

            @pl.when(i + 2 < steps)
            def _():
                wait_row(q2)
                issue_gather(q2)

            @pl.when(i >= 2)
            def _():
                wait_scatter(b2, q2)

            wait_ea(b2)
            compute(q, b2)
            wait_col(q)
            issue_scatter(b2, q)

            @pl.when(i + 3 < steps)
            def _():
                issue_row(i + 3, q3)

            @pl.when(i + 2 < steps)
            def _():
                issue_ea(i + 2, b2)
                issue_col(i + 2, q2)
        return _
    lax.fori_loop(0, steps // 4, _quad, None)

    wait_scatter(0, 2)
    wait_scatter(1, 3)

    plsc.subcore_barrier()

    off = c * _NP + s * _RPT
    pltpu.sync_copy(shared.at[pl.ds(s * _RPT, _RPT)],
                    out_hbm.at[pl.ds(off, _RPT)])


def _sc_edge(xa, ea, row_pad, col_pad):
    mesh = plsc.VectorSubcoreMesh(core_axis_name="c", subcore_axis_name="s")
    kern = functools.partial(
        pl.kernel,
        out_type=jax.ShapeDtypeStruct((_NC * _NP, _WIDE), jnp.float32),
        mesh=mesh,
        scratch_types=(
            [pltpu.VMEM_SHARED((_NP, _WIDE), jnp.float32)]
            + [pltpu.VMEM((_CHUNK,), jnp.int32)] * 4
            + [pltpu.VMEM((_CHUNK // 2, 2 * _H), jnp.float32)] * 2
            + [pltpu.VMEM((_CHUNK, _H), jnp.float32)] * 4
            + [pltpu.VMEM((_CHUNK, _WIDE), jnp.float32)] * 2
            + [pltpu.VMEM((_CHUNK,), jnp.int32)] * 4
            + [pltpu.SemaphoreType.DMA] * 18
        ),
        compiler_params=pltpu.CompilerParams(use_tc_tiling_on_sc=False),
    )(_sc_edge_body)
    return kern(xa, ea, row_pad, col_pad)


_BN = 1024


def _node_body(x_ref, h0_ref, h1_ref, bt_ref, u_ref, w1b_ref, b1b_ref,
               w2ax_ref, w2am_ref, w2au_ref, b2a_ref, w2b_ref, b2b_ref,
               o_ref):
    h0 = h0_ref[...]
    h1 = h1_ref[...]
    hsum = h0[:, :_H] + h1[:, :_H]
    cnt = h0[:, _H:_H + 1] + h1[:, _H:_H + 1]
    inv = 1.0 / jnp.maximum(cnt, 1.0)
    meanh = hsum * inv
    ind = (cnt > 0.0).astype(jnp.float32)

    wc = jnp.dot(w1b_ref[...], w2am_ref[...],
                 preferred_element_type=jnp.float32)
    bc = jnp.dot(b1b_ref[...], w2am_ref[...],
                 preferred_element_type=jnp.float32)
    ub = jnp.dot(u_ref[...], w2au_ref[...],
                 preferred_element_type=jnp.float32)

    onehot = (bt_ref[...] == lax.broadcasted_iota(jnp.int32, (1, _B), 1)
              ).astype(jnp.float32)

    z = (jnp.dot(x_ref[...], w2ax_ref[...],
                 preferred_element_type=jnp.float32)
         + jnp.dot(meanh, wc, preferred_element_type=jnp.float32)
         + ind * bc
         + jnp.dot(onehot, ub, preferred_element_type=jnp.float32)
         + b2a_ref[...])
    o_ref[...] = jnp.dot(jnp.maximum(z, 0.0), w2b_ref[...],
                         preferred_element_type=jnp.float32) + b2b_ref[...]


def _node(x_pad, h0, h1, bt_col, u, w1b, b1b_row, w2a_x, w2a_m, w2a_u,
          b2a_row, w2b, b2b_row):
    grid = _NP // _BN
    full = lambda shape: pl.BlockSpec(shape, lambda i: tuple(0 for _ in shape))
    return pl.pallas_call(
        _node_body,
        grid=(grid,),
        in_specs=[
            pl.BlockSpec((_BN, _D), lambda i: (i, 0)),
            pl.BlockSpec((_BN, _WIDE), lambda i: (i, 0)),
            pl.BlockSpec((_BN, _WIDE), lambda i: (i, 0)),
            pl.BlockSpec((_BN, 1), lambda i: (i, 0)),
            full((_B, _DU)),
            full((_H, _H)),
            full((1, _H)),
            full((_D, _H)),
            full((_H, _H)),
            full((_DU, _H)),
            full((1, _H)),
            full((_H, _DOUT)),
            full((1, _DOUT)),
        ],
        out_specs=pl.BlockSpec((_BN, _DOUT), lambda i: (i, 0)),
        out_shape=jax.ShapeDtypeStruct((_NP, _DOUT), jnp.float32),
    )(x_pad, h0, h1, bt_col, u, w1b, b1b_row, w2a_x, w2a_m, w2a_u,
      b2a_row, w2b, b2b_row)


def kernel(x, edge_index, edge_attr, u, batch,
           W1a, b1a, W1b, b1b, W2a, b2a, W2b, b2b):
    x_pad = jnp.pad(x, ((0, _NP - _N), (0, 0)))
    row_pad = jnp.pad(edge_index[0], (0, _EP - _E))
    col_pad = jnp.pad(edge_index[1], (0, _EP - _E), constant_values=_N)
    bt_col = jnp.pad(batch, (0, _NP - _N)).reshape(_NP, 1)

    xa = _prep_xa(x_pad, W1a[:_D])
    attr_t = jnp.pad(edge_attr.T, ((0, 0), (0, _EP - _E)))
    ea = _prep_ea(attr_t, W1a[_D:], b1a.reshape(1, _H))
    hp = _sc_edge(xa, ea, row_pad, col_pad)

    out_pad = _node(x_pad, hp[:_NP], hp[_NP:], bt_col, u,
                    W1b, b1b.reshape(1, _H),
                    W2a[:_D], W2a[_D:_D + _H], W2a[_D + _H:],
                    b2a.reshape(1, _H), W2b, b2b.reshape(1, _DOUT))
    return out_pad[:_N]
